# Initial kernel scaffold; baseline (speedup 1.0000x reference)
import jax
import jax.numpy as jnp
from jax.experimental import pallas as pl


def kernel(x, edge_index, bond_dist, lg_edge_index, angle, atom_W, atom_b, atom_ls, atom_lb, edge_W1, edge_b1, edge_l1s, edge_l1b, edge_W2, edge_b2, edge_l2s, edge_l2b, tri_W1, tri_b1, tri_l1s, tri_l1b, tri_W2, tri_b2, tri_l2s, tri_l2b, conv_W, conv_b, conv_ls, conv_lb, fc1_W, fc1_b, fc1_ls, fc1_lb, fc2_W, fc2_b, fc2_ls, fc2_lb, fc3_W, fc3_b):
    raise NotImplementedError("write your pallas kernel here")


# speedup vs baseline: 1.0275x; 1.0275x over previous
import functools

import jax
import jax.numpy as jnp
from jax import lax
from jax.experimental import pallas as pl
from jax.experimental.pallas import tpu as pltpu

H = 256
BLK = 1000


def _ln_silu(v, s, t):
    mu = jnp.mean(v, axis=-1, keepdims=True)
    var = jnp.mean((v - mu) ** 2, axis=-1, keepdims=True)
    v = (v - mu) * lax.rsqrt(var + 1e-5) * s + t
    return v * jax.nn.sigmoid(v)


def _mm_ln_silu(x, W, b, s, t):
    M, K = x.shape
    N = W.shape[1]

    def body(x_ref, w_ref, b_ref, s_ref, t_ref, o_ref):
        v = jnp.dot(x_ref[...], w_ref[...], preferred_element_type=jnp.float32)
        o_ref[...] = _ln_silu(v + b_ref[...], s_ref[...], t_ref[...])

    return pl.pallas_call(
        body,
        grid=(M // BLK,),
        in_specs=[
            pl.BlockSpec((BLK, K), lambda i: (i, 0)),
            pl.BlockSpec((K, N), lambda i: (0, 0)),
            pl.BlockSpec((1, N), lambda i: (0, 0)),
            pl.BlockSpec((1, N), lambda i: (0, 0)),
            pl.BlockSpec((1, N), lambda i: (0, 0)),
        ],
        out_specs=pl.BlockSpec((BLK, N), lambda i: (i, 0)),
        out_shape=jax.ShapeDtypeStruct((M, N), jnp.float32),
    )(x, W, b.reshape(1, -1), s.reshape(1, -1), t.reshape(1, -1))


def _rbf_mlp2(d, vmin, vmax, bins, W1, b1, s1, t1, W2, b2, s2, t2):
    M = d.shape[0]
    K1 = 128
    N1 = W1.shape[1]
    N2 = W2.shape[1]
    gamma = 1.0 / ((vmax - vmin) / bins) ** 2
    step = (vmax - vmin) / (bins - 1)

    def body(d_ref, w1_ref, b1_ref, s1_ref, t1_ref, w2_ref, b2_ref, s2_ref,
             t2_ref, o_ref):
        c = lax.broadcasted_iota(jnp.int32, (1, K1), 1).astype(jnp.float32)
        c = c * step + vmin
        r = jnp.exp(-gamma * (d_ref[...] - c) ** 2)
        v = jnp.dot(r, w1_ref[...], preferred_element_type=jnp.float32)
        v = _ln_silu(v + b1_ref[...], s1_ref[...], t1_ref[...])
        v = jnp.dot(v, w2_ref[...], preferred_element_type=jnp.float32)
        o_ref[...] = _ln_silu(v + b2_ref[...], s2_ref[...], t2_ref[...])

    W1p = jnp.zeros((K1, N1), jnp.float32).at[:bins].set(W1)
    return pl.pallas_call(
        body,
        grid=(M // BLK,),
        in_specs=[
            pl.BlockSpec((BLK, 1), lambda i: (i, 0)),
            pl.BlockSpec((K1, N1), lambda i: (0, 0)),
            pl.BlockSpec((1, N1), lambda i: (0, 0)),
            pl.BlockSpec((1, N1), lambda i: (0, 0)),
            pl.BlockSpec((1, N1), lambda i: (0, 0)),
            pl.BlockSpec((N1, N2), lambda i: (0, 0)),
            pl.BlockSpec((1, N2), lambda i: (0, 0)),
            pl.BlockSpec((1, N2), lambda i: (0, 0)),
            pl.BlockSpec((1, N2), lambda i: (0, 0)),
        ],
        out_specs=pl.BlockSpec((BLK, N2), lambda i: (i, 0)),
        out_shape=jax.ShapeDtypeStruct((M, N2), jnp.float32),
    )(d.reshape(M, 1), W1p, b1.reshape(1, -1), s1.reshape(1, -1),
      t1.reshape(1, -1), W2, b2.reshape(1, -1), s2.reshape(1, -1),
      t2.reshape(1, -1))


def _proj(h, Wg, bg):
    M = h.shape[0]

    def body(x_ref, w_ref, b_ref, o03_ref, o1_ref, o4_ref):
        v = jnp.dot(x_ref[...], w_ref[...], preferred_element_type=jnp.float32)
        v = v + b_ref[...]
        o03_ref[...] = v[:, :2 * H]
        o1_ref[...] = v[:, 2 * H:3 * H]
        o4_ref[...] = v[:, 3 * H:]

    return pl.pallas_call(
        body,
        grid=(M // BLK,),
        in_specs=[
            pl.BlockSpec((BLK, H), lambda i: (i, 0)),
            pl.BlockSpec((H, 4 * H), lambda i: (0, 0)),
            pl.BlockSpec((1, 4 * H), lambda i: (0, 0)),
        ],
        out_specs=[
            pl.BlockSpec((BLK, 2 * H), lambda i: (i, 0)),
            pl.BlockSpec((BLK, H), lambda i: (i, 0)),
            pl.BlockSpec((BLK, H), lambda i: (i, 0)),
        ],
        out_shape=[
            jax.ShapeDtypeStruct((M, 2 * H), jnp.float32),
            jax.ShapeDtypeStruct((M, H), jnp.float32),
            jax.ShapeDtypeStruct((M, H), jnp.float32),
        ],
    )(h, Wg, bg.reshape(1, -1))


def _mm(x, W, b):
    M, K = x.shape
    N = W.shape[1]

    def body(x_ref, w_ref, b_ref, o_ref):
        o_ref[...] = jnp.dot(x_ref[...], w_ref[...],
                             preferred_element_type=jnp.float32) + b_ref[...]

    return pl.pallas_call(
        body,
        grid=(M // BLK,),
        in_specs=[
            pl.BlockSpec((BLK, K), lambda i: (i, 0)),
            pl.BlockSpec((K, N), lambda i: (0, 0)),
            pl.BlockSpec((1, N), lambda i: (0, 0)),
        ],
        out_specs=pl.BlockSpec((BLK, N), lambda i: (i, 0)),
        out_shape=jax.ShapeDtypeStruct((M, N), jnp.float32),
    )(x, W, b.reshape(1, -1))


def _post_add(h, p, a, s, t):
    M = h.shape[0]

    def body(h_ref, p_ref, a_ref, s_ref, t_ref, o_ref):
        o_ref[...] = h_ref[...] + _ln_silu(p_ref[...] + a_ref[...],
                                           s_ref[...], t_ref[...])

    return pl.pallas_call(
        body,
        grid=(M // BLK,),
        in_specs=[
            pl.BlockSpec((BLK, H), lambda i: (i, 0)),
            pl.BlockSpec((BLK, H), lambda i: (i, 0)),
            pl.BlockSpec((BLK, H), lambda i: (i, 0)),
            pl.BlockSpec((1, H), lambda i: (0, 0)),
            pl.BlockSpec((1, H), lambda i: (0, 0)),
        ],
        out_specs=pl.BlockSpec((BLK, H), lambda i: (i, 0)),
        out_shape=jax.ShapeDtypeStruct((M, H), jnp.float32),
    )(h, p, a, s.reshape(1, -1), t.reshape(1, -1))


def _post(e, y, s, t):
    M = e.shape[0]

    def body(e_ref, y_ref, s_ref, t_ref, o_ref):
        o_ref[...] = e_ref[...] + _ln_silu(y_ref[...], s_ref[...], t_ref[...])

    return pl.pallas_call(
        body,
        grid=(M // BLK,),
        in_specs=[
            pl.BlockSpec((BLK, H), lambda i: (i, 0)),
            pl.BlockSpec((BLK, H), lambda i: (i, 0)),
            pl.BlockSpec((1, H), lambda i: (0, 0)),
            pl.BlockSpec((1, H), lambda i: (0, 0)),
        ],
        out_specs=pl.BlockSpec((BLK, H), lambda i: (i, 0)),
        out_shape=jax.ShapeDtypeStruct((M, H), jnp.float32),
    )(e, y, s.reshape(1, -1), t.reshape(1, -1))


def _readout(h, fc1_W, fc1_b, fc1_ls, fc1_lb, fc2_W, fc2_b, fc2_ls, fc2_lb,
             fc3_W, fc3_b):
    M = h.shape[0]

    def sum_body(h_ref, o_ref):
        @pl.when(pl.program_id(0) == 0)
        def _():
            o_ref[...] = jnp.zeros_like(o_ref)
        o_ref[...] += jnp.sum(h_ref[...], axis=0, keepdims=True)

    hsum = pl.pallas_call(
        sum_body,
        grid=(M // BLK,),
        in_specs=[pl.BlockSpec((BLK, H), lambda i: (i, 0))],
        out_specs=pl.BlockSpec((1, H), lambda i: (0, 0)),
        out_shape=jax.ShapeDtypeStruct((1, H), jnp.float32),
    )(h)

    def mlp_body(hs_ref, w1_ref, b1_ref, s1_ref, t1_ref, w2_ref, b2_ref,
                 s2_ref, t2_ref, w3_ref, b3_ref, o_ref):
        def ln_relu(v, s, t):
            mu = jnp.mean(v, axis=-1, keepdims=True)
            var = jnp.mean((v - mu) ** 2, axis=-1, keepdims=True)
            return jnp.maximum((v - mu) * lax.rsqrt(var + 1e-5) * s + t, 0.0)

        hg = hs_ref[...] * (1.0 / M)
        o = ln_relu(jnp.dot(hg, w1_ref[...],
                            preferred_element_type=jnp.float32) + b1_ref[...],
                    s1_ref[...], t1_ref[...])
        o = ln_relu(jnp.dot(o, w2_ref[...],
                            preferred_element_type=jnp.float32) + b2_ref[...],
                    s2_ref[...], t2_ref[...])
        o_ref[...] = jnp.dot(o, w3_ref[...],
                             preferred_element_type=jnp.float32) + b3_ref[...]

    fc3_Wp = jnp.zeros((128, 128), jnp.float32).at[:, :1].set(fc3_W)
    fc3_bp = jnp.zeros((128,), jnp.float32).at[:1].set(fc3_b)
    return pl.pallas_call(
        mlp_body,
        in_specs=[pl.BlockSpec(x.shape, lambda: tuple(0 for _ in x.shape))
                  for x in (hsum, fc1_W, fc1_b.reshape(1, -1),
                            fc1_ls.reshape(1, -1), fc1_lb.reshape(1, -1),
                            fc2_W, fc2_b.reshape(1, -1), fc2_ls.reshape(1, -1),
                            fc2_lb.reshape(1, -1), fc3_Wp,
                            fc3_bp.reshape(1, -1))],
        out_specs=pl.BlockSpec((1, 128), lambda: (0, 0)),
        out_shape=jax.ShapeDtypeStruct((1, 128), jnp.float32),
    )(hsum, fc1_W, fc1_b.reshape(1, -1), fc1_ls.reshape(1, -1),
      fc1_lb.reshape(1, -1), fc2_W, fc2_b.reshape(1, -1),
      fc2_ls.reshape(1, -1), fc2_lb.reshape(1, -1), fc3_Wp,
      fc3_bp.reshape(1, -1))


def _aggregate(P03, P1, Ce, src, dst, n):
    A = P03[src]
    y = A[:, :H] + P1[dst] + Ce
    sig = jax.nn.sigmoid(y)
    sh = sig * A[:, H:]
    sum_sh = jax.ops.segment_sum(sh, dst, num_segments=n)
    sum_s = jax.ops.segment_sum(sig, dst, num_segments=n)
    return sum_sh / (sum_s + 1e-6), y


def _egconv(h, e, src, dst, n, Wl, bl, lsl, lbl):
    Wg = jnp.concatenate([Wl[0], Wl[3], Wl[1], Wl[4]], axis=1)
    bg = jnp.concatenate([bl[0], bl[3], bl[1], bl[4]], axis=0)
    P03, P1, P4 = _proj(h, Wg, bg)
    Ce = _mm(e, Wl[2], bl[2])
    agg, y = _aggregate(P03, P1, Ce, src, dst, n)
    h_new = _post_add(h, P4, agg, lsl[0], lbl[0])
    e_new = _post(e, y, lsl[1], lbl[1])
    return h_new, e_new


def kernel(x, edge_index, bond_dist, lg_edge_index, angle, atom_W, atom_b,
           atom_ls, atom_lb, edge_W1, edge_b1, edge_l1s, edge_l1b, edge_W2,
           edge_b2, edge_l2s, edge_l2b, tri_W1, tri_b1, tri_l1s, tri_l1b,
           tri_W2, tri_b2, tri_l2s, tri_l2b, conv_W, conv_b, conv_ls, conv_lb,
           fc1_W, fc1_b, fc1_ls, fc1_lb, fc2_W, fc2_b, fc2_ls, fc2_lb, fc3_W,
           fc3_b):
    N = x.shape[0]
    E = bond_dist.shape[0]
    src, dst = edge_index[0], edge_index[1]
    lsrc, ldst = lg_edge_index[0], lg_edge_index[1]

    xp = jnp.zeros((N, 128), jnp.float32).at[:, :x.shape[1]].set(x)
    atom_Wp = jnp.zeros((128, H), jnp.float32).at[:x.shape[1]].set(atom_W)
    h = _mm_ln_silu(xp, atom_Wp, atom_b, atom_ls, atom_lb)
    e = _rbf_mlp2(bond_dist, 0.0, 8.0, 80, edge_W1, edge_b1, edge_l1s,
                  edge_l1b, edge_W2, edge_b2, edge_l2s, edge_l2b)
    z = _rbf_mlp2(angle, -1.0, 1.0, 40, tri_W1, tri_b1, tri_l1s, tri_l1b,
                  tri_W2, tri_b2, tri_l2s, tri_l2b)

    c = 0
    for _ in range(2):
        e, z = _egconv(e, z, lsrc, ldst, E, conv_W[c], conv_b[c], conv_ls[c],
                       conv_lb[c]); c += 1
        h, e = _egconv(h, e, src, dst, N, conv_W[c], conv_b[c], conv_ls[c],
                       conv_lb[c]); c += 1
    for _ in range(2):
        h, e = _egconv(h, e, src, dst, N, conv_W[c], conv_b[c], conv_ls[c],
                       conv_lb[c]); c += 1

    o = _readout(h, fc1_W, fc1_b, fc1_ls, fc1_lb, fc2_W, fc2_b, fc2_ls,
                 fc2_lb, fc3_W, fc3_b)
    return o[0:1, 0:1]
